# baseline (speedup 1.0000x reference)
import jax
import jax.numpy as jnp
from jax import lax
from jax.experimental import pallas as pl
from jax.experimental.pallas import tpu as pltpu
from jax.experimental.pallas import tpu_sc as plsc

DIM = 128
NW = 32
BATCH = 4096
FIELDS = 26
ROWS_PER_W = BATCH // NW
NB = 4
CHUNK = NB * FIELDS
NCHUNK = ROWS_PER_W // NB


def _emb_body(table_hbm, idx_hbm, out_hbm, idx_v, buf0, buf1,
              sg0, sg1, so0, so1):
    wid = lax.axis_index("s") * 2 + lax.axis_index("c")
    row0 = wid * ROWS_PER_W
    pltpu.sync_copy(idx_hbm.at[wid], idx_v)
    bufs = (buf0, buf1)
    sg = (sg0, sg1)
    so = (so0, so1)

    def gather(c, b):
        return pltpu.make_async_copy(table_hbm.at[idx_v.at[c]], bufs[b], sg[b])

    def out_copy(c, b, j):
        return pltpu.make_async_copy(
            bufs[b].at[pl.ds(j * FIELDS, FIELDS)],
            out_hbm.at[row0 + c * NB + j],
            so[b],
        )

    gather(0, 0).start()

    def body(g, carry):
        for b in range(2):
            c = g * 2 + b
            @pl.when(c > 0)
            def _():
                for j in range(NB):
                    out_copy(c - 1, 1 - b, j).wait()
            @pl.when(c + 1 < NCHUNK)
            def _():
                gather(c + 1, 1 - b).start()
            gather(c, b).wait()
            for j in range(NB):
                out_copy(c, b, j).start()
        return carry

    lax.fori_loop(0, NCHUNK // 2, body, 0)
    for j in range(NB):
        out_copy(NCHUNK - 1, (NCHUNK - 1) % 2, j).wait()


def kernel(x, table):
    b, f = x.shape
    idx3 = x.reshape(-1).astype(jnp.int32).reshape(NW, NCHUNK, CHUNK)
    mesh = plsc.VectorSubcoreMesh(core_axis_name="c", subcore_axis_name="s")
    k = pl.kernel(
        _emb_body,
        mesh=mesh,
        compiler_params=pltpu.CompilerParams(use_tc_tiling_on_sc=True),
        out_type=jax.ShapeDtypeStruct((BATCH, FIELDS, DIM), jnp.float32),
        scratch_types=[
            pltpu.VMEM((NCHUNK, CHUNK), jnp.int32),
            pltpu.VMEM((CHUNK, DIM), jnp.float32),
            pltpu.VMEM((CHUNK, DIM), jnp.float32),
            pltpu.SemaphoreType.DMA,
            pltpu.SemaphoreType.DMA,
            pltpu.SemaphoreType.DMA,
            pltpu.SemaphoreType.DMA,
        ],
    )
    return k(table, idx3)


# speedup vs baseline: 1.8428x; 1.8428x over previous
import jax
import jax.numpy as jnp
from jax import lax
from jax.experimental import pallas as pl
from jax.experimental.pallas import tpu as pltpu
from jax.experimental.pallas import tpu_sc as plsc

DIM = 128
CHUNK = 128
NW = 32
BATCH = 4096
FIELDS = 26
TOTAL = BATCH * FIELDS
PER_W = TOTAL // NW
NCHUNK = PER_W // CHUNK


def _emb_body(table_hbm, idx_hbm, out_hbm, idx_v, buf0, buf1,
              sg0, sg1, so0, so1):
    wid = lax.axis_index("s") * 2 + lax.axis_index("c")
    base = wid * PER_W
    pltpu.sync_copy(idx_hbm.at[wid], idx_v)
    bufs = (buf0, buf1)
    sg = (sg0, sg1)
    so = (so0, so1)

    def gather(c, b):
        return pltpu.make_async_copy(table_hbm.at[idx_v.at[c]], bufs[b], sg[b])

    def out_copy(c, b):
        return pltpu.make_async_copy(
            bufs[b], out_hbm.at[pl.ds(base + c * CHUNK, CHUNK)], so[b])

    gather(0, 0).start()

    def body(g, carry):
        for b in range(2):
            c = g * 2 + b
            @pl.when(c > 0)
            def _():
                out_copy(c - 1, 1 - b).wait()
            @pl.when(c + 1 < NCHUNK)
            def _():
                gather(c + 1, 1 - b).start()
            gather(c, b).wait()
            out_copy(c, b).start()
        return carry

    lax.fori_loop(0, NCHUNK // 2, body, 0)
    out_copy(NCHUNK - 1, (NCHUNK - 1) % 2).wait()


def kernel(x, table):
    b, f = x.shape
    flat = x.T.reshape(-1).astype(jnp.int32)
    idx3 = flat.reshape(NW, NCHUNK, CHUNK)
    mesh = plsc.VectorSubcoreMesh(core_axis_name="c", subcore_axis_name="s")
    k = pl.kernel(
        _emb_body,
        mesh=mesh,
        out_type=jax.ShapeDtypeStruct((TOTAL, DIM), jnp.float32),
        scratch_types=[
            pltpu.VMEM((NCHUNK, CHUNK), jnp.int32),
            pltpu.VMEM((CHUNK, DIM), jnp.float32),
            pltpu.VMEM((CHUNK, DIM), jnp.float32),
            pltpu.SemaphoreType.DMA,
            pltpu.SemaphoreType.DMA,
            pltpu.SemaphoreType.DMA,
            pltpu.SemaphoreType.DMA,
        ],
    )
    out = k(table, idx3)
    return out.reshape(f, b, DIM).transpose(1, 0, 2)


# speedup vs baseline: 1.8470x; 1.0023x over previous
import jax
import jax.numpy as jnp
from jax import lax
from jax.experimental import pallas as pl
from jax.experimental.pallas import tpu as pltpu
from jax.experimental.pallas import tpu_sc as plsc

DIM = 128
CHUNK = 128
NW = 32
BATCH = 4096
FIELDS = 26
TOTAL = BATCH * FIELDS
PER_W = TOTAL // NW
NCHUNK = PER_W // CHUNK


NBUF = 4


def _emb_body(table_hbm, idx_hbm, out_hbm, idx_v,
              buf0, buf1, buf2, buf3,
              sg0, sg1, sg2, sg3, so0, so1, so2, so3):
    wid = lax.axis_index("s") * 2 + lax.axis_index("c")
    base = wid * PER_W
    pltpu.sync_copy(idx_hbm.at[wid], idx_v)
    bufs = (buf0, buf1, buf2, buf3)
    sg = (sg0, sg1, sg2, sg3)
    so = (so0, so1, so2, so3)

    def gather(c):
        b = c % NBUF
        return pltpu.make_async_copy(table_hbm.at[idx_v.at[c]], bufs[b], sg[b])

    def out_copy(c):
        b = c % NBUF
        return pltpu.make_async_copy(
            bufs[b], out_hbm.at[pl.ds(base + c * CHUNK, CHUNK)], so[b])

    for c in range(NBUF - 1):
        gather(c).start()
    for c in range(NCHUNK):
        if c >= 1:
            out_copy(c - 1).wait()
        if c + NBUF - 1 < NCHUNK:
            gather(c + NBUF - 1).start()
        gather(c).wait()
        out_copy(c).start()
    out_copy(NCHUNK - 1).wait()


def kernel(x, table):
    b, f = x.shape
    flat = x.T.reshape(-1).astype(jnp.int32)
    idx3 = flat.reshape(NW, NCHUNK, CHUNK)
    mesh = plsc.VectorSubcoreMesh(core_axis_name="c", subcore_axis_name="s")
    k = pl.kernel(
        _emb_body,
        mesh=mesh,
        out_type=jax.ShapeDtypeStruct((TOTAL, DIM), jnp.float32),
        scratch_types=(
            [pltpu.VMEM((NCHUNK, CHUNK), jnp.int32)]
            + [pltpu.VMEM((CHUNK, DIM), jnp.float32)] * NBUF
            + [pltpu.SemaphoreType.DMA] * (2 * NBUF)
        ),
    )
    out = k(table, idx3)
    return out.reshape(f, b, DIM).transpose(1, 0, 2)
